# baseline (speedup 1.0000x reference)
import jax
import jax.numpy as jnp
from jax import lax
from jax.experimental import pallas as pl
from jax.experimental.pallas import tpu as pltpu
from jax.experimental.pallas import tpu_sc as plsc

B = 16384
D = 64
NC = 2
NS = 16
NW = NC * NS
BPW = B // NW
CHUNK = 128
NCHUNK = BPW // CHUNK
GROUPS = BPW // 16


def _rsqrt(x):
    i = plsc.bitcast(x, jnp.int32)
    i = jnp.int32(0x5F3759DF) - (i >> 1)
    y = plsc.bitcast(i, jnp.float32)
    for _ in range(3):
        y = y * (jnp.float32(1.5) - jnp.float32(0.5) * x * y * y)
    return y


def _scale(sq):
    return jnp.minimum(jnp.float32(1.0),
                       _rsqrt(jnp.maximum(sq, jnp.float32(1e-14))))


def _body(uid_hbm, iid_hbm, users_hbm, items_hbm, out_hbm,
          uidx_v, iidx_v, urows_v, irows_v, out_v, sem):
    w = lax.axis_index("s") * NC + lax.axis_index("c")
    crow = w * NCHUNK
    pltpu.sync_copy(uid_hbm.at[pl.ds(crow, NCHUNK)], uidx_v)
    pltpu.sync_copy(iid_hbm.at[pl.ds(crow, NCHUNK)], iidx_v)
    copies = []
    for j in range(NCHUNK):
        copies.append(pltpu.async_copy(
            users_hbm.at[uidx_v.at[j]],
            urows_v.at[pl.ds(j * CHUNK, CHUNK)], sem))
        copies.append(pltpu.async_copy(
            items_hbm.at[iidx_v.at[j]],
            irows_v.at[pl.ds(j * CHUNK, CHUNK)], sem))
    for c in copies:
        c.wait()

    lanes = lax.iota(jnp.int32, 16)

    def group(g, carry):
        row_vec = g * 16 + lanes
        zero = jnp.zeros((16,), jnp.float32)
        uu, ii, ui = zero, zero, zero
        for d in range(D):
            d_vec = jnp.full((16,), d, jnp.int32)
            u = plsc.load_gather(urows_v, [row_vec, d_vec])
            v = plsc.load_gather(irows_v, [row_vec, d_vec])
            uu = uu + u * u
            ii = ii + v * v
            ui = ui + u * v
        p = ui * _scale(uu) * _scale(ii)
        out_v[pl.ds(g * 16, 16)] = (
            jnp.float32(1.0) / (jnp.float32(1.0) + jnp.exp(-p)))
        return carry

    lax.fori_loop(0, GROUPS, group, 0)
    pltpu.sync_copy(out_v, out_hbm.at[pl.ds(w * BPW, BPW)])


def kernel(user_ids, item_ids, users, items):
    uid2 = user_ids.reshape(B // CHUNK, CHUNK).astype(jnp.int32)
    iid2 = item_ids.reshape(B // CHUNK, CHUNK).astype(jnp.int32)
    run = pl.kernel(
        _body,
        out_type=jax.ShapeDtypeStruct((B,), jnp.float32),
        mesh=plsc.VectorSubcoreMesh(core_axis_name="c", subcore_axis_name="s"),
        compiler_params=pltpu.CompilerParams(
            needs_layout_passes=False, use_tc_tiling_on_sc=False),
        scratch_types=[
            pltpu.VMEM((NCHUNK, CHUNK), jnp.int32),
            pltpu.VMEM((NCHUNK, CHUNK), jnp.int32),
            pltpu.VMEM((BPW, D), jnp.float32),
            pltpu.VMEM((BPW, D), jnp.float32),
            pltpu.VMEM((BPW,), jnp.float32),
            pltpu.SemaphoreType.DMA,
        ],
    )
    return run(uid2, iid2, users, items)
